# Initial kernel scaffold; baseline (speedup 1.0000x reference)
import jax
import jax.numpy as jnp
from jax.experimental import pallas as pl


def kernel(edge_logits, distances, edge_index, num_nodes):
    raise NotImplementedError("write your pallas kernel here")


# speedup vs baseline: 2.9195x; 2.9195x over previous
import functools

import jax
import jax.numpy as jnp
from jax import lax
from jax.experimental import pallas as pl
from jax.experimental.pallas import tpu as pltpu
from jax.experimental.pallas import tpu_sc as plsc

_NC = 2
_NS = 16
_L = 16
_NW = _NC * _NS

_N = 4096
_E = 262144
_EW = _E // _NW
_CH = 128
_NCH = _EW // _CH
_VPC = _CH // _L
_LAMBDA = 2.0
_ZCH = _N // _NS


def _sc_body(dist_hbm, src_hbm, dst_hbm, logit_hbm, hist_out, pd_out,
             srcv, dstv, logv, idxv, gathv, probv, pdv, zv, hist_sh, sem):
    c = lax.axis_index("c")
    s = lax.axis_index("s")
    wid = s * _NC + c

    pltpu.async_copy(src_hbm.at[wid], srcv, sem)
    pltpu.async_copy(dst_hbm.at[wid], dstv, sem)
    pltpu.async_copy(logit_hbm.at[wid], logv, sem)

    def _zbody(i, _):
        zv[pl.ds(i * _L, _L)] = jnp.zeros((_L,), jnp.float32)
        return 0
    lax.fori_loop(0, _ZCH // _L, _zbody, 0)

    pltpu.make_async_copy(src_hbm.at[wid], srcv, sem).wait()
    pltpu.make_async_copy(dst_hbm.at[wid], dstv, sem).wait()
    pltpu.make_async_copy(logit_hbm.at[wid], logv, sem).wait()

    pltpu.sync_copy(zv, hist_sh.at[pl.ds(s * _ZCH, _ZCH)])
    plsc.subcore_barrier()

    def _ibody(i, _):
        j = i // _VPC
        o = (i % _VPC) * _L
        sv = srcv[j, pl.ds(o, _L)]
        dv = dstv[j, pl.ds(o, _L)]
        idxv[pl.ds(i * _L, _L)] = sv * _N + dv
        x = logv[j, pl.ds(o, _L)]
        probv[j, pl.ds(o, _L)] = 1.0 / (1.0 + jnp.exp(-x))
        return 0
    lax.fori_loop(0, _EW // _L, _ibody, 0)

    gather = pltpu.async_copy(dist_hbm.at[idxv], gathv, sem)

    def _sbody(j, _):
        pltpu.sync_copy(probv.at[j], hist_sh.at[srcv.at[j]], add=True)
        return 0
    lax.fori_loop(0, _NCH, _sbody, 0)

    gather.wait()

    def _dbody(i, acc):
        j = i // _VPC
        o = (i % _VPC) * _L
        return acc + probv[j, pl.ds(o, _L)] * gathv[pl.ds(i * _L, _L)]
    acc = lax.fori_loop(0, _EW // _L, _dbody, jnp.zeros((_L,), jnp.float32))
    pdv[...] = acc
    pltpu.sync_copy(pdv, pd_out.at[wid])

    plsc.subcore_barrier()

    @pl.when(s == 0)
    def _():
        pltpu.sync_copy(hist_sh, hist_out.at[c])


_sc_call = functools.partial(
    pl.kernel,
    out_type=[
        jax.ShapeDtypeStruct((_NC, _N), jnp.float32),
        jax.ShapeDtypeStruct((_NW, _L), jnp.float32),
    ],
    mesh=plsc.VectorSubcoreMesh(
        core_axis_name="c", subcore_axis_name="s",
        num_cores=_NC, num_subcores=_NS),
    scratch_types=[
        pltpu.VMEM((_NCH, _CH), jnp.int32),
        pltpu.VMEM((_NCH, _CH), jnp.int32),
        pltpu.VMEM((_NCH, _CH), jnp.float32),
        pltpu.VMEM((_EW,), jnp.int32),
        pltpu.VMEM((_EW,), jnp.float32),
        pltpu.VMEM((_NCH, _CH), jnp.float32),
        pltpu.VMEM((_L,), jnp.float32),
        pltpu.VMEM((_ZCH,), jnp.float32),
        pltpu.VMEM_SHARED((_N,), jnp.float32),
        pltpu.SemaphoreType.DMA,
    ],
)(_sc_body)


def _tc_reduce(h_ref, pd_ref, out_ref):
    wd = jnp.sum(h_ref[...], axis=0, keepdims=True)
    d = wd - 2.0
    loss_deg = jnp.sum(d * d) * (1.0 / _N)
    loss_dist = jnp.sum(pd_ref[...])
    out_ref[0, 0] = loss_dist + _LAMBDA * loss_deg


_tc_call = pl.pallas_call(
    _tc_reduce,
    out_shape=jax.ShapeDtypeStruct((1, 1), jnp.float32),
    out_specs=pl.BlockSpec(memory_space=pltpu.SMEM),
)


def kernel(edge_logits, distances, edge_index, num_nodes):
    del num_nodes
    src = edge_index[0].astype(jnp.int32).reshape(_NW, _NCH, _CH)
    dst = edge_index[1].astype(jnp.int32).reshape(_NW, _NCH, _CH)
    logits = edge_logits.reshape(_NW, _NCH, _CH)
    dist_flat = distances.reshape(-1)
    hist, pd = _sc_call(dist_flat, src, dst, logits)
    res = _tc_call(hist, pd)
    return res[0, 0]


# speedup vs baseline: 6.4723x; 2.2169x over previous
import functools

import jax
import jax.numpy as jnp
from jax import lax
from jax.experimental import pallas as pl
from jax.experimental.pallas import tpu as pltpu
from jax.experimental.pallas import tpu_sc as plsc

_NC = 2
_NS = 16
_L = 16
_NW = _NC * _NS

_N = 4096
_E = 262144
_EW = _E // _NW
_CH = 128
_NCH = _EW // _CH
_VPC = _CH // _L
_LAMBDA = 2.0
_ZCH = _N // _NS


def _sc_body(dist_hbm, src_hbm, dst_hbm, logit_hbm, hist_out, pd_out,
             srcv, dstv, logv, idxv, gathv, probv, pdv, zv, hist_sh, sem):
    c = lax.axis_index("c")
    s = lax.axis_index("s")
    wid = s * _NC + c

    pltpu.async_copy(src_hbm.at[wid], srcv, sem)
    pltpu.async_copy(dst_hbm.at[wid], dstv, sem)
    pltpu.async_copy(logit_hbm.at[wid], logv, sem)

    def _zbody(i, _):
        zv[pl.ds(i * _L, _L)] = jnp.zeros((_L,), jnp.float32)
        return 0
    lax.fori_loop(0, _ZCH // _L, _zbody, 0)

    pltpu.make_async_copy(src_hbm.at[wid], srcv, sem).wait()
    pltpu.make_async_copy(dst_hbm.at[wid], dstv, sem).wait()
    pltpu.make_async_copy(logit_hbm.at[wid], logv, sem).wait()

    pltpu.sync_copy(zv, hist_sh.at[pl.ds(s * _ZCH, _ZCH)])
    plsc.subcore_barrier()

    def _ibody(i, _):
        j = i // _VPC
        o = (i % _VPC) * _L
        sv = srcv[j, pl.ds(o, _L)]
        dv = dstv[j, pl.ds(o, _L)]
        phys = ((sv >> 3) << 15) + ((dv >> 7) << 10) + ((sv & 7) << 7) + (dv & 127)
        idxv[pl.ds(i * _L, _L)] = phys
        x = logv[j, pl.ds(o, _L)]
        probv[j, pl.ds(o, _L)] = 1.0 / (1.0 + jnp.exp(-x))
        return 0
    lax.fori_loop(0, _EW // _L, _ibody, 0)

    gather = pltpu.async_copy(dist_hbm.at[idxv], gathv, sem)

    def _sbody(j, _):
        pltpu.sync_copy(probv.at[j], hist_sh.at[srcv.at[j]], add=True)
        return 0
    lax.fori_loop(0, _NCH, _sbody, 0)

    gather.wait()

    def _dbody(i, acc):
        j = i // _VPC
        o = (i % _VPC) * _L
        return acc + probv[j, pl.ds(o, _L)] * gathv[pl.ds(i * _L, _L)]
    acc = lax.fori_loop(0, _EW // _L, _dbody, jnp.zeros((_L,), jnp.float32))
    pdv[...] = acc
    pltpu.sync_copy(pdv, pd_out.at[wid])

    plsc.subcore_barrier()

    @pl.when(s == 0)
    def _():
        pltpu.sync_copy(hist_sh, hist_out.at[c])


_sc_call = functools.partial(
    pl.kernel,
    out_type=[
        jax.ShapeDtypeStruct((_NC, _N), jnp.float32),
        jax.ShapeDtypeStruct((_NW, _L), jnp.float32),
    ],
    mesh=plsc.VectorSubcoreMesh(
        core_axis_name="c", subcore_axis_name="s",
        num_cores=_NC, num_subcores=_NS),
    scratch_types=[
        pltpu.VMEM((_NCH, _CH), jnp.int32),
        pltpu.VMEM((_NCH, _CH), jnp.int32),
        pltpu.VMEM((_NCH, _CH), jnp.float32),
        pltpu.VMEM((_EW,), jnp.int32),
        pltpu.VMEM((_EW,), jnp.float32),
        pltpu.VMEM((_NCH, _CH), jnp.float32),
        pltpu.VMEM((_L,), jnp.float32),
        pltpu.VMEM((_ZCH,), jnp.float32),
        pltpu.VMEM_SHARED((_N,), jnp.float32),
        pltpu.SemaphoreType.DMA,
    ],
)(_sc_body)


def _tc_reduce(h_ref, pd_ref, out_ref):
    wd = jnp.sum(h_ref[...], axis=0, keepdims=True)
    d = wd - 2.0
    loss_deg = jnp.sum(d * d) * (1.0 / _N)
    loss_dist = jnp.sum(pd_ref[...])
    out_ref[0, 0] = loss_dist + _LAMBDA * loss_deg


_tc_call = pl.pallas_call(
    _tc_reduce,
    out_shape=jax.ShapeDtypeStruct((1, 1), jnp.float32),
    out_specs=pl.BlockSpec(memory_space=pltpu.SMEM),
)


def kernel(edge_logits, distances, edge_index, num_nodes):
    del num_nodes
    src = edge_index[0].astype(jnp.int32).reshape(_NW, _NCH, _CH)
    dst = edge_index[1].astype(jnp.int32).reshape(_NW, _NCH, _CH)
    logits = edge_logits.reshape(_NW, _NCH, _CH)
    dist_flat = (distances.reshape(_N // 8, 8, _N // 128, 128)
                 .transpose(0, 2, 1, 3).reshape(_N * _N))
    hist, pd = _sc_call(dist_flat, src, dst, logits)
    res = _tc_call(hist, pd)
    return res[0, 0]
